# Initial kernel scaffold; baseline (speedup 1.0000x reference)
import functools

import jax
import jax.numpy as jnp
from jax import lax
from jax.experimental import pallas as pl
from jax.experimental.pallas import tpu as pltpu
from jax.experimental.pallas import tpu_sc as plsc

_N = 10000
_D = 128
_NC = 2
_NS = 16
_NW = _NC * _NS

_N_PAD = 10240
_ROWS = _N_PAD // 128
_CHUNK = 128
_EDGE_CHUNKS = 79
_TILE_E = _CHUNK * _EDGE_CHUNKS
_E_PAD = _TILE_E * _NW
_RPT = _N_PAD // _NS


def _sc_mesh():
    return plsc.VectorSubcoreMesh(core_axis_name="c", subcore_axis_name="s")


def _deg_body(dst_hbm, deg_hbm, idx_v, hist, zbuf, rowid, degacc):
    c = lax.axis_index("c")
    s = lax.axis_index("s")
    wid = c * _NS + s
    zero16 = jnp.zeros((16,), jnp.float32)
    ones16 = jnp.ones((16,), jnp.float32)
    iota16 = lax.broadcasted_iota(jnp.int32, (16,), 0)

    def zh(i, carry):
        hist[i // 8, pl.ds((i % 8) * 16, 16)] = zero16
        return carry

    lax.fori_loop(0, _ROWS * 8, zh, 0)

    def zz(i, carry):
        zbuf[i // 8, pl.ds((i % 8) * 16, 16)] = zero16
        return carry

    lax.fori_loop(0, (_ROWS // _NS) * 8, zz, 0)

    def ri(j, carry):
        rowid[pl.ds(j * 16, 16)] = iota16 + j * 16
        return carry

    lax.fori_loop(0, _ROWS // 16, ri, 0)

    pltpu.sync_copy(zbuf, degacc.at[pl.ds(s * (_ROWS // _NS), _ROWS // _NS)])
    plsc.subcore_barrier()

    def chunk_body(ci, carry):
        base = wid * _TILE_E + ci * _CHUNK
        pltpu.sync_copy(dst_hbm.at[pl.ds(base, _CHUNK)], idx_v)

        def lane_body(j, inner):
            idx = idx_v[pl.ds(j * 16, 16)]
            plsc.addupdate_scatter(hist, [idx >> 7, idx & 127], ones16)
            return inner

        lax.fori_loop(0, _CHUNK // 16, lane_body, 0)
        return carry

    lax.fori_loop(0, _EDGE_CHUNKS, chunk_body, 0)

    pltpu.sync_copy(hist, degacc.at[rowid], add=True)
    plsc.subcore_barrier()
    pltpu.sync_copy(degacc.at[pl.ds(s * (_ROWS // _NS), _ROWS // _NS)],
                    deg_hbm.at[c, pl.ds(s * (_ROWS // _NS), _ROWS // _NS)])


def _deg_call(dst_pad):
    return pl.kernel(
        _deg_body,
        out_type=jax.ShapeDtypeStruct((_NC, _ROWS, 128), jnp.float32),
        mesh=_sc_mesh(),
        scratch_types=[
            pltpu.VMEM((_CHUNK,), jnp.int32),
            pltpu.VMEM((_ROWS, 128), jnp.float32),
            pltpu.VMEM((_ROWS // _NS, 128), jnp.float32),
            pltpu.VMEM((_ROWS,), jnp.int32),
            pltpu.VMEM_SHARED((_ROWS, 128), jnp.float32),
        ],
    )(dst_pad)


def _agg_body(src_hbm, dst_hbm, y_hbm, out_hbm, idx_s, idx_d, rows, sem, acc):
    c = lax.axis_index("c")
    s = lax.axis_index("s")
    wid = c * _NS + s
    zero16 = jnp.zeros((16,), jnp.float32)

    def zr(i, carry):
        rows[i // 8, pl.ds((i % 8) * 16, 16)] = zero16
        return carry

    lax.fori_loop(0, _CHUNK * 8, zr, 0)
    for k in range(_RPT // _CHUNK):
        pltpu.sync_copy(rows, acc.at[pl.ds(s * _RPT + k * _CHUNK, _CHUNK)])
    plsc.subcore_barrier()

    def chunk_body(ci, carry):
        base = wid * _TILE_E + ci * _CHUNK
        pltpu.sync_copy(src_hbm.at[pl.ds(base, _CHUNK)], idx_s)
        pltpu.sync_copy(dst_hbm.at[pl.ds(base, _CHUNK)], idx_d)
        pltpu.async_copy(y_hbm.at[idx_s], rows, sem).wait()
        pltpu.sync_copy(rows, acc.at[idx_d], add=True)
        return carry

    lax.fori_loop(0, _EDGE_CHUNKS, chunk_body, 0)
    plsc.subcore_barrier()
    pltpu.sync_copy(acc.at[pl.ds(s * _RPT, _RPT)],
                    out_hbm.at[c, pl.ds(s * _RPT, _RPT)])


def _agg_call(src_pad, dst_pad, y):
    return pl.kernel(
        _agg_body,
        out_type=jax.ShapeDtypeStruct((_NC, _N_PAD, _D), jnp.float32),
        mesh=_sc_mesh(),
        scratch_types=[
            pltpu.VMEM((_CHUNK,), jnp.int32),
            pltpu.VMEM((_CHUNK,), jnp.int32),
            pltpu.VMEM((_CHUNK, _D), jnp.float32),
            pltpu.SemaphoreType.DMA,
            pltpu.VMEM_SHARED((_N_PAD, _D), jnp.float32),
        ],
    )(src_pad, dst_pad, y)


def _tca_body(degp_ref, x_ref, w_ref, y_ref, dinv_ref):
    deg = degp_ref[0] + degp_ref[1] + 1.0
    dinv = lax.rsqrt(deg)
    xw = jnp.dot(x_ref[...], w_ref[...], preferred_element_type=jnp.float32)
    y_ref[...] = xw * dinv
    dinv_ref[...] = dinv


def _tca_call(degp, x_pad, w1):
    return pl.pallas_call(
        _tca_body,
        grid=(_ROWS,),
        in_specs=[
            pl.BlockSpec((_NC, 128, 1), lambda i: (0, i, 0)),
            pl.BlockSpec((128, _D), lambda i: (i, 0)),
            pl.BlockSpec((_D, _D), lambda i: (0, 0)),
        ],
        out_specs=[
            pl.BlockSpec((128, _D), lambda i: (i, 0)),
            pl.BlockSpec((128, 1), lambda i: (i, 0)),
        ],
        out_shape=[
            jax.ShapeDtypeStruct((_N_PAD, _D), jnp.float32),
            jax.ShapeDtypeStruct((_N_PAD, 1), jnp.float32),
        ],
    )(degp, x_pad, w1)


def _tcb_body(a_ref, y1_ref, dinv_ref, b_ref, w_ref, y2_ref):
    t = (a_ref[0] + a_ref[1] + y1_ref[...]) * dinv_ref[...] + b_ref[...]
    t = jnp.maximum(t, 0.0)
    y2_ref[...] = jnp.dot(t, w_ref[...],
                          preferred_element_type=jnp.float32) * dinv_ref[...]


def _tcb_call(a1, y1, dinv, b1, w2):
    return pl.pallas_call(
        _tcb_body,
        grid=(_ROWS,),
        in_specs=[
            pl.BlockSpec((_NC, 128, _D), lambda i: (0, i, 0)),
            pl.BlockSpec((128, _D), lambda i: (i, 0)),
            pl.BlockSpec((128, 1), lambda i: (i, 0)),
            pl.BlockSpec((1, _D), lambda i: (0, 0)),
            pl.BlockSpec((_D, _D), lambda i: (0, 0)),
        ],
        out_specs=pl.BlockSpec((128, _D), lambda i: (i, 0)),
        out_shape=jax.ShapeDtypeStruct((_N_PAD, _D), jnp.float32),
    )(a1, y1, dinv, b1, w2)


def _tcc_body(a_ref, y2_ref, dinv_ref, b_ref, o_ref):
    t = (a_ref[0] + a_ref[1] + y2_ref[...]) * dinv_ref[...] + b_ref[...]
    o_ref[...] = jnp.maximum(t, 0.0)


def _tcc_call(a2, y2, dinv, b2):
    return pl.pallas_call(
        _tcc_body,
        grid=(_ROWS,),
        in_specs=[
            pl.BlockSpec((_NC, 128, _D), lambda i: (0, i, 0)),
            pl.BlockSpec((128, _D), lambda i: (i, 0)),
            pl.BlockSpec((128, 1), lambda i: (i, 0)),
            pl.BlockSpec((1, _D), lambda i: (0, 0)),
        ],
        out_specs=pl.BlockSpec((128, _D), lambda i: (i, 0)),
        out_shape=jax.ShapeDtypeStruct((_N_PAD, _D), jnp.float32),
    )(a2, y2, dinv, b2)


def kernel(x, edge_index, W1, b1, W2, b2):
    src = edge_index[0]
    dst = edge_index[1]
    n_fill = _E_PAD - src.shape[0]
    fill = jnp.full((n_fill,), _N_PAD - 1, dtype=jnp.int32)
    src_p = jnp.concatenate([src, fill])
    dst_p = jnp.concatenate([dst, fill])
    x_p = jnp.pad(x, ((0, _N_PAD - x.shape[0]), (0, 0)))

    degp = _deg_call(dst_p).reshape(_NC, _N_PAD, 1)
    y1, dinv = _tca_call(degp, x_p, W1)
    a1 = _agg_call(src_p, dst_p, y1)
    y2 = _tcb_call(a1, y1, dinv, b1.reshape(1, _D), W2)
    a2 = _agg_call(src_p, dst_p, y2)
    out = _tcc_call(a2, y2, dinv, b2.reshape(1, _D))
    return out[: x.shape[0]]


# speedup vs baseline: 8.9041x; 8.9041x over previous
import functools

import jax
import jax.numpy as jnp
from jax import lax
from jax.experimental import pallas as pl
from jax.experimental.pallas import tpu as pltpu
from jax.experimental.pallas import tpu_sc as plsc

_N = 10000
_D = 128
_NC = 2
_NS = 16
_NW = _NC * _NS

_N_PAD = 10240
_ROWS = _N_PAD // 128
_CHUNK = 128
_EDGE_CHUNKS = 79
_TILE_E = _CHUNK * _EDGE_CHUNKS
_E_PAD = _TILE_E * _NW
_RPT = _N_PAD // _NS


def _sc_mesh():
    return plsc.VectorSubcoreMesh(core_axis_name="c", subcore_axis_name="s")


def _deg_body(dst_hbm, deg_hbm, idx_v, onerows, acc):
    c = lax.axis_index("c")
    s = lax.axis_index("s")
    wid = c * _NS + s
    zero16 = jnp.zeros((16,), jnp.float32)
    ones16 = jnp.ones((16,), jnp.float32)

    def zfill(i, carry):
        onerows[i // 8, pl.ds((i % 8) * 16, 16)] = zero16
        return carry

    lax.fori_loop(0, _CHUNK * 8, zfill, 0)
    for k in range(_RPT // _CHUNK):
        pltpu.sync_copy(onerows, acc.at[pl.ds(s * _RPT + k * _CHUNK, _CHUNK)])

    def ofill(i, carry):
        onerows[i // 8, pl.ds((i % 8) * 16, 16)] = ones16
        return carry

    lax.fori_loop(0, _CHUNK * 8, ofill, 0)
    plsc.subcore_barrier()

    def chunk_body(ci, carry):
        base = wid * _TILE_E + ci * _CHUNK
        pltpu.sync_copy(dst_hbm.at[pl.ds(base, _CHUNK)], idx_v)
        pltpu.sync_copy(onerows, acc.at[idx_v], add=True)
        return carry

    lax.fori_loop(0, _EDGE_CHUNKS, chunk_body, 0)
    plsc.subcore_barrier()
    pltpu.sync_copy(acc.at[pl.ds(s * _RPT, _RPT)],
                    deg_hbm.at[c, pl.ds(s * _RPT, _RPT)])


def _deg_call(dst_pad):
    return pl.kernel(
        _deg_body,
        out_type=jax.ShapeDtypeStruct((_NC, _N_PAD, _D), jnp.float32),
        mesh=_sc_mesh(),
        scratch_types=[
            pltpu.VMEM((_CHUNK,), jnp.int32),
            pltpu.VMEM((_CHUNK, _D), jnp.float32),
            pltpu.VMEM_SHARED((_N_PAD, _D), jnp.float32),
        ],
    )(dst_pad)


def _agg_body(src_hbm, dst_hbm, y_hbm, out_hbm, idx_s, idx_d, rows, sem, acc):
    c = lax.axis_index("c")
    s = lax.axis_index("s")
    wid = c * _NS + s
    zero16 = jnp.zeros((16,), jnp.float32)

    def zr(i, carry):
        rows[i // 8, pl.ds((i % 8) * 16, 16)] = zero16
        return carry

    lax.fori_loop(0, _CHUNK * 8, zr, 0)
    for k in range(_RPT // _CHUNK):
        pltpu.sync_copy(rows, acc.at[pl.ds(s * _RPT + k * _CHUNK, _CHUNK)])
    plsc.subcore_barrier()

    def chunk_body(ci, carry):
        base = wid * _TILE_E + ci * _CHUNK
        pltpu.sync_copy(src_hbm.at[pl.ds(base, _CHUNK)], idx_s)
        pltpu.sync_copy(dst_hbm.at[pl.ds(base, _CHUNK)], idx_d)
        pltpu.async_copy(y_hbm.at[idx_s], rows, sem).wait()
        pltpu.sync_copy(rows, acc.at[idx_d], add=True)
        return carry

    lax.fori_loop(0, _EDGE_CHUNKS, chunk_body, 0)
    plsc.subcore_barrier()
    pltpu.sync_copy(acc.at[pl.ds(s * _RPT, _RPT)],
                    out_hbm.at[c, pl.ds(s * _RPT, _RPT)])


def _agg_call(src_pad, dst_pad, y):
    return pl.kernel(
        _agg_body,
        out_type=jax.ShapeDtypeStruct((_NC, _N_PAD, _D), jnp.float32),
        mesh=_sc_mesh(),
        scratch_types=[
            pltpu.VMEM((_CHUNK,), jnp.int32),
            pltpu.VMEM((_CHUNK,), jnp.int32),
            pltpu.VMEM((_CHUNK, _D), jnp.float32),
            pltpu.SemaphoreType.DMA,
            pltpu.VMEM_SHARED((_N_PAD, _D), jnp.float32),
        ],
    )(src_pad, dst_pad, y)


def _tca_body(degp_ref, x_ref, w_ref, y_ref, dinv_ref):
    deg = degp_ref[0, :, 0:1] + degp_ref[1, :, 0:1] + 1.0
    dinv = lax.rsqrt(deg)
    xw = jnp.dot(x_ref[...], w_ref[...], preferred_element_type=jnp.float32)
    y_ref[...] = xw * dinv
    dinv_ref[...] = dinv


def _tca_call(degp, x_pad, w1):
    return pl.pallas_call(
        _tca_body,
        grid=(_ROWS,),
        in_specs=[
            pl.BlockSpec((_NC, 128, _D), lambda i: (0, i, 0)),
            pl.BlockSpec((128, _D), lambda i: (i, 0)),
            pl.BlockSpec((_D, _D), lambda i: (0, 0)),
        ],
        out_specs=[
            pl.BlockSpec((128, _D), lambda i: (i, 0)),
            pl.BlockSpec((128, 1), lambda i: (i, 0)),
        ],
        out_shape=[
            jax.ShapeDtypeStruct((_N_PAD, _D), jnp.float32),
            jax.ShapeDtypeStruct((_N_PAD, 1), jnp.float32),
        ],
    )(degp, x_pad, w1)


def _tcb_body(a_ref, y1_ref, dinv_ref, b_ref, w_ref, y2_ref):
    t = (a_ref[0] + a_ref[1] + y1_ref[...]) * dinv_ref[...] + b_ref[...]
    t = jnp.maximum(t, 0.0)
    y2_ref[...] = jnp.dot(t, w_ref[...],
                          preferred_element_type=jnp.float32) * dinv_ref[...]


def _tcb_call(a1, y1, dinv, b1, w2):
    return pl.pallas_call(
        _tcb_body,
        grid=(_ROWS,),
        in_specs=[
            pl.BlockSpec((_NC, 128, _D), lambda i: (0, i, 0)),
            pl.BlockSpec((128, _D), lambda i: (i, 0)),
            pl.BlockSpec((128, 1), lambda i: (i, 0)),
            pl.BlockSpec((1, _D), lambda i: (0, 0)),
            pl.BlockSpec((_D, _D), lambda i: (0, 0)),
        ],
        out_specs=pl.BlockSpec((128, _D), lambda i: (i, 0)),
        out_shape=jax.ShapeDtypeStruct((_N_PAD, _D), jnp.float32),
    )(a1, y1, dinv, b1, w2)


def _tcc_body(a_ref, y2_ref, dinv_ref, b_ref, o_ref):
    t = (a_ref[0] + a_ref[1] + y2_ref[...]) * dinv_ref[...] + b_ref[...]
    o_ref[...] = jnp.maximum(t, 0.0)


def _tcc_call(a2, y2, dinv, b2):
    return pl.pallas_call(
        _tcc_body,
        grid=(_ROWS,),
        in_specs=[
            pl.BlockSpec((_NC, 128, _D), lambda i: (0, i, 0)),
            pl.BlockSpec((128, _D), lambda i: (i, 0)),
            pl.BlockSpec((128, 1), lambda i: (i, 0)),
            pl.BlockSpec((1, _D), lambda i: (0, 0)),
        ],
        out_specs=pl.BlockSpec((128, _D), lambda i: (i, 0)),
        out_shape=jax.ShapeDtypeStruct((_N_PAD, _D), jnp.float32),
    )(a2, y2, dinv, b2)


def kernel(x, edge_index, W1, b1, W2, b2):
    src = edge_index[0]
    dst = edge_index[1]
    n_fill = _E_PAD - src.shape[0]
    fill = jnp.full((n_fill,), _N_PAD - 1, dtype=jnp.int32)
    src_p = jnp.concatenate([src, fill])
    dst_p = jnp.concatenate([dst, fill])
    x_p = jnp.pad(x, ((0, _N_PAD - x.shape[0]), (0, 0)))

    degp = _deg_call(dst_p)
    y1, dinv = _tca_call(degp, x_p, W1)
    a1 = _agg_call(src_p, dst_p, y1)
    y2 = _tcb_call(a1, y1, dinv, b1.reshape(1, _D), W2)
    a2 = _agg_call(src_p, dst_p, y2)
    out = _tcc_call(a2, y2, dinv, b2.reshape(1, _D))
    return out[: x.shape[0]]
